# baseline (speedup 1.0000x reference)
import functools

import jax
import jax.numpy as jnp
from jax import lax
from jax.experimental import pallas as pl
from jax.experimental.pallas import tpu as pltpu

B, C, H, W = 4, 192, 384, 384
E = 16
CH = C // 4
ROWS = B * C
COLS = H * W
BLK = 4096
NSTEP = COLS // BLK


def _reduce_body(x_ref, o_ref):
    @pl.when(pl.program_id(0) == 0)
    def _():
        o_ref[...] = jnp.zeros_like(o_ref)

    blk = x_ref[...]
    part = blk.reshape(ROWS, BLK // 128, 128).sum(axis=1)
    o_ref[...] += part


def _mlp_body(p_ref, w1_ref, b1_ref, w2_ref, b2_ref, o_ref):
    pooled = p_ref[...].sum(axis=1) * (1.0 / COLS)
    pooled = pooled.reshape(B, C)
    h = lax.dot_general(pooled, w1_ref[...],
                        (((1,), (1,)), ((), ())),
                        preferred_element_type=jnp.float32)
    h = jnp.maximum(h + b1_ref[...], 0.0)
    logits = lax.dot_general(h, w2_ref[...],
                             (((1,), (1,)), ((), ())),
                             preferred_element_type=jnp.float32)
    logits = logits + b2_ref[...]
    m = jnp.max(logits, axis=1, keepdims=True)
    e = jnp.exp(logits - m)
    o_ref[...] = e / jnp.sum(e, axis=1, keepdims=True)


@jax.jit
def kernel(x, w1, b1, w2, b2):
    x2 = x.reshape(ROWS, COLS)
    partials = pl.pallas_call(
        _reduce_body,
        grid=(NSTEP,),
        in_specs=[pl.BlockSpec((ROWS, BLK), lambda i: (0, i))],
        out_specs=pl.BlockSpec((ROWS, 128), lambda i: (0, 0)),
        out_shape=jax.ShapeDtypeStruct((ROWS, 128), jnp.float32),
    )(x2)
    out = pl.pallas_call(
        _mlp_body,
        out_shape=jax.ShapeDtypeStruct((B, E), jnp.float32),
    )(partials, w1, b1.reshape(1, CH), w2, b2.reshape(1, E))
    return out
